# Initial kernel scaffold; baseline (speedup 1.0000x reference)
import jax
import jax.numpy as jnp
from jax.experimental import pallas as pl


def kernel(home_x, away_x, home_edge_attr, away_edge_attr, home_features, away_features, WA, bA, WB, bB, WC, bC, Wfc, bfc, home_edge_index, away_edge_index, window_size):
    raise NotImplementedError("write your pallas kernel here")


# speedup vs baseline: 69.5305x; 69.5305x over previous
import functools

import jax
import jax.numpy as jnp
from jax import lax
from jax.experimental import pallas as pl
from jax.experimental.pallas import tpu as pltpu
from jax.experimental.pallas import tpu_sc as plsc

N = 100000
E = 1600000
NP = 100096
BLK = NP // 16
PB = 2 * NP * 16 // 128
PBLK = PB // 8
EROWS = 12544
TILE_EROWS = EROWS // 16
EPAD = EROWS * 128
R = 4
NCH = TILE_EROWS // R

_SC_PARAMS = pltpu.CompilerParams(use_tc_tiling_on_sc=False)

_GDN = lax.GatherDimensionNumbers(
    offset_dims=(), collapsed_slice_dims=(0,), start_index_map=(0,))


def _splat(vec16, lane):
    idx = jnp.full((16, 1), lane, dtype=jnp.int32)
    return lax.gather(vec16, idx, _GDN, (1,),
                      mode=lax.GatherScatterMode.PROMISE_IN_BOUNDS)


def _sc_degree(dst3, ew3, zrow):
    mesh = plsc.VectorSubcoreMesh(core_axis_name="c", subcore_axis_name="s")

    RD = 8
    NCHD = TILE_EROWS // RD

    @functools.partial(
        pl.kernel,
        out_type=jax.ShapeDtypeStruct((2 * NP,), jnp.float32),
        mesh=mesh,
        compiler_params=_SC_PARAMS,
        scratch_types=[
            pltpu.VMEM((2, RD, 128), jnp.int32),
            pltpu.VMEM((2, RD, 128), jnp.float32),
            pltpu.VMEM_SHARED((NP,), jnp.float32),
            pltpu.SemaphoreType.DMA((2,)),
        ],
    )
    def k(dst_h, ew_h, z_h, out_h, dstb, ewb, acc, sem_l):
        c = lax.axis_index("c")
        s = lax.axis_index("s")
        base = s * TILE_EROWS

        def lin(i, p):
            r0 = base + i * RD
            return [
                pltpu.make_async_copy(dst_h.at[c, pl.ds(r0, RD)], dstb.at[p],
                                      sem_l.at[p]),
                pltpu.make_async_copy(ew_h.at[c, pl.ds(r0, RD)], ewb.at[p],
                                      sem_l.at[p]),
            ]

        for d in lin(0, 0):
            d.start()
        for d in lin(1, 1):
            d.start()
        pltpu.sync_copy(z_h, acc.at[pl.ds(s * BLK, BLK)])
        plsc.subcore_barrier()

        def chunk(i, p):
            for d in lin(i, p):
                d.wait()
            for r in range(RD):
                pltpu.sync_copy(ewb.at[p, r], acc.at[dstb.at[p, r]], add=True)

            @pl.when(i + 2 <= NCHD - 1)
            def _():
                for d in lin(i + 2, p):
                    d.start()

        def sbody(sb, carry):
            chunk(sb * 2, 0)
            chunk(sb * 2 + 1, 1)
            return carry

        lax.fori_loop(0, NCHD // 2, sbody, 0)
        plsc.subcore_barrier()
        pltpu.sync_copy(acc.at[pl.ds(s * BLK, BLK)],
                        out_h.at[pl.ds(c * NP + s * BLK, BLK)])

    return k(dst3, ew3, zrow)


def _sc_propagate(gtab, src3, dst3, ew3, zrows):
    mesh = plsc.VectorSubcoreMesh(core_axis_name="c", subcore_axis_name="s")

    @functools.partial(
        pl.kernel,
        out_type=jax.ShapeDtypeStruct((2 * NP, 16), jnp.float32),
        mesh=mesh,
        compiler_params=_SC_PARAMS,
        scratch_types=[
            pltpu.VMEM((2, R, 128), jnp.int32),
            pltpu.VMEM((2, R, 128), jnp.int32),
            pltpu.VMEM((2, R, 128), jnp.float32),
            pltpu.VMEM((2, R * 128, 16), jnp.float32),
            pltpu.VMEM((R * 128, 16), jnp.float32),
            pltpu.VMEM_SHARED((NP, 16), jnp.float32),
            pltpu.SemaphoreType.DMA((2,)),
            pltpu.SemaphoreType.DMA((2,)),
            pltpu.SemaphoreType.DMA,
        ],
    )
    def k(g_h, src_h, dst_h, ew_h, z_h, out_h,
          srcb, dstb, ewb, rows, rout, acc, sem_l, sem_g, sem_s):
        c = lax.axis_index("c")
        s = lax.axis_index("s")
        base = s * TILE_EROWS

        def lin(i, p):
            r0 = base + i * R
            return [
                pltpu.make_async_copy(src_h.at[c, pl.ds(r0, R)], srcb.at[p],
                                      sem_l.at[p]),
                pltpu.make_async_copy(dst_h.at[c, pl.ds(r0, R)], dstb.at[p],
                                      sem_l.at[p]),
                pltpu.make_async_copy(ew_h.at[c, pl.ds(r0, R)], ewb.at[p],
                                      sem_l.at[p]),
            ]

        def gat(p):
            return [pltpu.make_async_copy(g_h.at[srcb.at[p, r]],
                                          rows.at[p, pl.ds(r * 128, 128)],
                                          sem_g.at[p]) for r in range(R)]

        def sca_start(p, r):
            pltpu.async_copy(rout.at[pl.ds(r * 128, 128)],
                             acc.at[dstb.at[p, r]], sem_s, add=True)

        def sca_wait(p, r):
            pltpu.make_async_copy(rout.at[pl.ds(r * 128, 128)],
                                  acc.at[dstb.at[p, r]], sem_s).wait()

        for d in lin(0, 0):
            d.start()
        for d in lin(1, 1):
            d.start()
        pltpu.sync_copy(z_h, acc.at[pl.ds(s * BLK, BLK)])
        plsc.subcore_barrier()
        for d in lin(0, 0):
            d.wait()
        for d in gat(0):
            d.start()

        def chunk(i, p, pn):
            for d in gat(p):
                d.wait()

            @pl.when(i + 1 <= NCH - 1)
            def _():
                for d in lin(i + 1, pn):
                    d.wait()
                for d in gat(pn):
                    d.start()

            @pl.when(i >= 1)
            def _():
                for r in range(R):
                    sca_wait(pn, r)

            for r in range(R):
                for j2 in range(8):
                    w16 = ewb[p, r, pl.ds(j2 * 16, 16)]
                    for l in range(16):
                        rr = r * 128 + j2 * 16 + l
                        rout[rr, :] = rows[p, rr, :] * _splat(w16, l)
                sca_start(p, r)

            @pl.when(i + 2 <= NCH - 1)
            def _():
                for d in lin(i + 2, p):
                    d.start()

        def sbody(sb, carry):
            chunk(sb * 2, 0, 1)
            chunk(sb * 2 + 1, 1, 0)
            return carry

        lax.fori_loop(0, NCH // 2, sbody, 0)
        for r in range(R):
            sca_wait(1, r)
        plsc.subcore_barrier()
        pltpu.sync_copy(acc.at[pl.ds(s * BLK, BLK)],
                        out_h.at[pl.ds(c * NP + s * BLK, BLK)])

    return k(gtab, src3, dst3, ew3, zrows)


def _dinv_block(deg8, e816, i):
    deg = jnp.dot(deg8, e816, preferred_element_type=jnp.float32)
    r = lax.broadcasted_iota(jnp.int32, deg.shape, 0)
    cc = lax.broadcasted_iota(jnp.int32, deg.shape, 1)
    nid = (i * PBLK + r) * 8 + cc // 16
    valid = (nid % NP) < N
    return jnp.where(valid, lax.rsqrt(deg + 1.0), 0.0), valid


def _tc_pre(x16, Z, deg8, e816, WA128, WB128):
    def f(x_ref, z_ref, d8_ref, e_ref, wa_ref, wb_ref, o_ref):
        di, _ = _dinv_block(d8_ref[...], e_ref[...], pl.program_id(0))
        h = jnp.dot(x_ref[...], wa_ref[...], preferred_element_type=jnp.float32)
        h = h + jnp.dot(z_ref[...], wb_ref[...],
                        preferred_element_type=jnp.float32)
        o_ref[...] = h * di

    return pl.pallas_call(
        f, grid=(8,),
        in_specs=[
            pl.BlockSpec((PBLK, 128), lambda i: (i, 0)),
            pl.BlockSpec((PBLK, 128), lambda i: (i, 0)),
            pl.BlockSpec((PBLK, 8), lambda i: (i, 0)),
            pl.BlockSpec((8, 128), lambda i: (0, 0)),
            pl.BlockSpec((128, 128), lambda i: (0, 0)),
            pl.BlockSpec((128, 128), lambda i: (0, 0)),
        ],
        out_specs=pl.BlockSpec((PBLK, 128), lambda i: (i, 0)),
        out_shape=jax.ShapeDtypeStruct((PB, 128), jnp.float32),
    )(x16, Z, deg8, e816, WA128, WB128)


def _tc_mid(S1, g1, deg8, e816, b128, WC128):
    def f(s_ref, g_ref, d8_ref, e_ref, b_ref, wc_ref, z_ref, o_ref):
        di, _ = _dinv_block(d8_ref[...], e_ref[...], pl.program_id(0))
        z = jnp.maximum(di * (s_ref[...] + g_ref[...]) + b_ref[...], 0.0)
        z_ref[...] = z
        o_ref[...] = di * jnp.dot(z, wc_ref[...],
                                  preferred_element_type=jnp.float32)

    return pl.pallas_call(
        f, grid=(8,),
        in_specs=[
            pl.BlockSpec((PBLK, 128), lambda i: (i, 0)),
            pl.BlockSpec((PBLK, 128), lambda i: (i, 0)),
            pl.BlockSpec((PBLK, 8), lambda i: (i, 0)),
            pl.BlockSpec((8, 128), lambda i: (0, 0)),
            pl.BlockSpec((1, 128), lambda i: (0, 0)),
            pl.BlockSpec((128, 128), lambda i: (0, 0)),
        ],
        out_specs=[
            pl.BlockSpec((PBLK, 128), lambda i: (i, 0)),
            pl.BlockSpec((PBLK, 128), lambda i: (i, 0)),
        ],
        out_shape=[
            jax.ShapeDtypeStruct((PB, 128), jnp.float32),
            jax.ShapeDtypeStruct((PB, 128), jnp.float32),
        ],
    )(S1, g1, deg8, e816, b128, WC128)


def _tc_post(S2, g2, deg8, e816, b128):
    def f(s_ref, g_ref, d8_ref, e_ref, b_ref, o_ref):
        i = pl.program_id(0)
        di, valid = _dinv_block(d8_ref[...], e_ref[...], i)
        y = jnp.maximum(di * (s_ref[...] + g_ref[...]) + b_ref[...], 0.0)
        y = jnp.where(valid, y, 0.0)
        part = jnp.sum(y, axis=0, keepdims=True)

        @pl.when(i == 0)
        def _():
            o_ref[...] = jnp.zeros_like(o_ref)

        team = i // 4
        o_ref[pl.ds(team, 1), :] += part

    return pl.pallas_call(
        f, grid=(8,),
        in_specs=[
            pl.BlockSpec((PBLK, 128), lambda i: (i, 0)),
            pl.BlockSpec((PBLK, 128), lambda i: (i, 0)),
            pl.BlockSpec((PBLK, 8), lambda i: (i, 0)),
            pl.BlockSpec((8, 128), lambda i: (0, 0)),
            pl.BlockSpec((1, 128), lambda i: (0, 0)),
        ],
        out_specs=pl.BlockSpec((2, 128), lambda i: (0, 0)),
        out_shape=jax.ShapeDtypeStruct((2, 128), jnp.float32),
    )(S2, g2, deg8, e816, b128)


def _tc_fc(comb, Wfc, bfc):
    def f(c_ref, w_ref, b_ref, o_ref):
        o_ref[...] = lax.dot_general(
            c_ref[...], w_ref[...], (((1,), (1,)), ((), ())),
            preferred_element_type=jnp.float32) + b_ref[...]

    return pl.pallas_call(
        f, out_shape=jax.ShapeDtypeStruct((comb.shape[0], Wfc.shape[0]),
                                          jnp.float32))(comb, Wfc, bfc)


def kernel(home_x, away_x, home_edge_attr, away_edge_attr, home_features,
           away_features, WA, bA, WB, bB, WC, bC, Wfc, bfc, home_edge_index,
           away_edge_index, window_size):
    T = home_x.shape[0]

    X = jnp.concatenate([
        jnp.pad(home_x, ((0, 0), (0, NP - N), (0, 12))),
        jnp.pad(away_x, ((0, 0), (0, NP - N), (0, 12))),
    ], axis=1).reshape(T, PB, 128)

    pad = EPAD - E
    pidx = (jnp.arange(pad, dtype=jnp.int32) * 131) % N

    def prep(ei, ea, src_off):
        src = jnp.concatenate([ei[0], pidx + src_off]).reshape(EROWS, 128)
        dst = jnp.concatenate([ei[1], pidx]).reshape(EROWS, 128)
        ew = jnp.concatenate(
            [ea, jnp.zeros((pad,), jnp.float32)]).reshape(EROWS, 128)
        return src, dst, ew

    sh, dh, wh = prep(home_edge_index, home_edge_attr, 0)
    sa, da, wa = prep(away_edge_index, away_edge_attr, NP)
    src3 = jnp.stack([sh, sa])
    dst3 = jnp.stack([dh, da])
    ew3 = jnp.stack([wh, wa])

    zrow1 = jnp.zeros((BLK,), jnp.float32)
    zrow2 = jnp.zeros((BLK, 16), jnp.float32)

    eye8 = jnp.eye(8, dtype=jnp.float32)
    WA128 = jnp.kron(eye8, jnp.pad(WA, ((0, 12), (0, 0))))
    WB128 = jnp.kron(eye8, WB)
    WC128 = jnp.kron(eye8, WC)
    e816 = jnp.kron(eye8, jnp.ones((1, 16), jnp.float32))
    b1_128 = jnp.tile(bA + bB, 8).reshape(1, 128)
    bC_128 = jnp.tile(bC, 8).reshape(1, 128)

    deg = _sc_degree(dst3, ew3, zrow1)
    deg8 = deg.reshape(PB, 8)

    Z = jnp.zeros((PB, 128), jnp.float32)
    sums = []
    for t in range(T):
        g1 = _tc_pre(X[t], Z, deg8, e816, WA128, WB128)
        S1 = _sc_propagate(g1.reshape(2 * NP, 16), src3, dst3, ew3, zrow2)
        Z, g2 = _tc_mid(S1.reshape(PB, 128), g1, deg8, e816, b1_128, WC128)
        S2 = _sc_propagate(g2.reshape(2 * NP, 16), src3, dst3, ew3, zrow2)
        sums.append(_tc_post(S2.reshape(PB, 128), g2, deg8, e816, bC_128))

    means = jnp.stack(sums).reshape(T, 2, 8, 16).sum(axis=2) / N
    comb = jnp.concatenate(
        [means[:, 0, :], home_features, means[:, 1, :], away_features], axis=1)
    logits = _tc_fc(comb, Wfc, bfc.reshape(1, 3))
    return logits + jnp.zeros_like(logits) * window_size
